# baseline (speedup 1.0000x reference)
import jax
import jax.numpy as jnp
from jax import lax
from jax.experimental import pallas as pl
from jax.experimental.pallas import tpu as pltpu

N = 4096
D = 64
BRW = 2048
BK = 256
NRB = N // BRW
NK = N // BK


def _body(x_ref, pinv0_ref, pinv1_ref, phi0_ref, phi1_ref,
          w0_ref, w1_ref, k0_ref, k1_ref,
          out_ref, xp0_ref, xp1_ref, y0_ref, y1_ref):
    p = pl.program_id(0)
    rb = pl.program_id(1)
    kc = pl.program_id(2)

    @pl.when(jnp.logical_and(p == 0, jnp.logical_and(rb == 0, kc == 0)))
    def _():
        xp0_ref[...] = jnp.dot(x_ref[...], w0_ref[...],
                               preferred_element_type=jnp.float32)
        xp1_ref[...] = jnp.dot(x_ref[...], w1_ref[...],
                               preferred_element_type=jnp.float32)

    rows = pl.ds(rb * BRW, BRW)
    kcs = pl.ds(kc * BK, BK)

    def stage0(pinv_ref, xp_ref, y_ref, k_ref):
        part = jnp.dot(pinv_ref[...], xp_ref[kcs, :],
                       preferred_element_type=jnp.float32)
        acc = jnp.where(kc == 0, part, y_ref[rows, :] + part)
        y_ref[rows, :] = jnp.where(kc == NK - 1, k_ref[rows, :] * acc, acc)

    def stage1(phi_ref, y_ref):
        part = jnp.dot(phi_ref[...], y_ref[kcs, :],
                       preferred_element_type=jnp.float32)
        acc = jnp.where(kc == 0, part, out_ref[0, :, :] + part)
        out_ref[0, :, :] = jnp.where(kc == NK - 1, jnp.maximum(acc, 0.0), acc)

    @pl.when(p == 0)
    def _():
        stage0(pinv0_ref, xp0_ref, y0_ref, k0_ref)

    @pl.when(p == 1)
    def _():
        stage0(pinv1_ref, xp1_ref, y1_ref, k1_ref)

    @pl.when(p == 2)
    def _():
        stage1(phi0_ref, y0_ref)

    @pl.when(p == 3)
    def _():
        stage1(phi1_ref, y1_ref)


def kernel(x, phi_inv_0, phi_0, phi_inv_1, phi_1, W0, W1, k0, k1):
    def bmap(ap):
        def imap(p, rb, kc):
            r = jnp.where(p < ap, 0, jnp.where(p == ap, rb, NRB - 1))
            c = jnp.where(p < ap, 0, jnp.where(p == ap, kc, NK - 1))
            return (r, c)
        return imap

    def out_map(p, rb, kc):
        s = jnp.where(p >= 3, 1, 0)
        r = jnp.where(p >= 2, rb, 0)
        return (s, r, 0)

    full = lambda shape: pl.BlockSpec(shape, lambda p, rb, kc: (0,) * len(shape))

    return pl.pallas_call(
        _body,
        grid=(4, NRB, NK),
        in_specs=[
            full((N, D)),
            pl.BlockSpec((BRW, BK), bmap(0)),
            pl.BlockSpec((BRW, BK), bmap(1)),
            pl.BlockSpec((BRW, BK), bmap(2)),
            pl.BlockSpec((BRW, BK), bmap(3)),
            full((D, D)),
            full((D, D)),
            full((N, 1)),
            full((N, 1)),
        ],
        out_specs=pl.BlockSpec((1, BRW, D), out_map),
        out_shape=jax.ShapeDtypeStruct((2, N, D), jnp.float32),
        scratch_shapes=[pltpu.VMEM((N, D), jnp.float32),
                        pltpu.VMEM((N, D), jnp.float32),
                        pltpu.VMEM((N, D), jnp.float32),
                        pltpu.VMEM((N, D), jnp.float32)],
    )(x, phi_inv_0, phi_inv_1, phi_0, phi_1, W0, W1, k0, k1)


# speedup vs baseline: 1.3967x; 1.3967x over previous
import jax
import jax.numpy as jnp
from jax import lax
from jax.experimental import pallas as pl
from jax.experimental.pallas import tpu as pltpu

N = 4096
D = 64
BR = 128
NB = N // BR

_DN_NT = (((1,), (1,)), ((), ()))


def _body(x_ref, pinv0_ref, pinv1_ref, phi0_ref, phi1_ref,
          w0_ref, w1_ref, k0_ref, k1_ref,
          out_ref, xpt0_ref, xpt1_ref, yt0_ref, yt1_ref):
    p = pl.program_id(0)
    i = pl.program_id(1)

    @pl.when(jnp.logical_and(p == 0, i == 0))
    def _():
        xpt0_ref[...] = lax.dot_general(
            w0_ref[...], x_ref[...], (((0,), (1,)), ((), ())),
            preferred_element_type=jnp.float32)
        xpt1_ref[...] = lax.dot_general(
            w1_ref[...], x_ref[...], (((0,), (1,)), ((), ())),
            preferred_element_type=jnp.float32)

    @pl.when(p == 0)
    def _():
        t0 = lax.dot_general(xpt0_ref[...], pinv0_ref[...], _DN_NT,
                             preferred_element_type=jnp.float32)
        yt0_ref[:, pl.ds(i * BR, BR)] = k0_ref[:, pl.ds(i * BR, BR)] * t0
        t1 = lax.dot_general(xpt1_ref[...], pinv1_ref[...], _DN_NT,
                             preferred_element_type=jnp.float32)
        yt1_ref[:, pl.ds(i * BR, BR)] = k1_ref[:, pl.ds(i * BR, BR)] * t1

    @pl.when(p == 1)
    def _():
        z0 = lax.dot_general(yt0_ref[...], phi0_ref[...], _DN_NT,
                             preferred_element_type=jnp.float32)
        out_ref[0, :, :] = jnp.maximum(z0, 0.0).T
        z1 = lax.dot_general(yt1_ref[...], phi1_ref[...], _DN_NT,
                             preferred_element_type=jnp.float32)
        out_ref[1, :, :] = jnp.maximum(z1, 0.0).T


def kernel(x, phi_inv_0, phi_0, phi_inv_1, phi_1, W0, W1, k0, k1):
    def basis_map(active_phase):
        def imap(p, i):
            blk = jnp.where(p < active_phase, 0,
                            jnp.where(p == active_phase, i, NB - 1))
            return (blk, 0)
        return imap

    full = lambda shape: pl.BlockSpec(shape, lambda p, i: (0,) * len(shape))

    return pl.pallas_call(
        _body,
        grid=(2, NB),
        in_specs=[
            full((N, D)),
            pl.BlockSpec((BR, N), basis_map(0)),
            pl.BlockSpec((BR, N), basis_map(0)),
            pl.BlockSpec((BR, N), basis_map(1)),
            pl.BlockSpec((BR, N), basis_map(1)),
            full((D, D)),
            full((D, D)),
            full((1, N)),
            full((1, N)),
        ],
        out_specs=pl.BlockSpec((2, BR, D),
                               lambda p, i: (0, jnp.where(p == 1, i, 0), 0)),
        out_shape=jax.ShapeDtypeStruct((2, N, D), jnp.float32),
        scratch_shapes=[pltpu.VMEM((D, N), jnp.float32),
                        pltpu.VMEM((D, N), jnp.float32),
                        pltpu.VMEM((D, N), jnp.float32),
                        pltpu.VMEM((D, N), jnp.float32)],
    )(x, phi_inv_0, phi_inv_1, phi_0, phi_1, W0, W1,
      k0.reshape(1, N), k1.reshape(1, N))


# speedup vs baseline: 1.7480x; 1.2516x over previous
import jax
import jax.numpy as jnp
from jax import lax
from jax.experimental import pallas as pl
from jax.experimental.pallas import tpu as pltpu

N = 4096
D = 64
BR = 512
NB = N // BR

_DN_NT = (((1,), (1,)), ((), ()))


def _body(x_ref, pinv0_ref, pinv1_ref, phi0_ref, phi1_ref,
          w0_ref, w1_ref, k0_ref, k1_ref,
          out_ref, buf0_ref, buf1_ref, xpt0_ref, xpt1_ref,
          yt0_ref, yt1_ref, sem):
    p = pl.program_id(0)
    i = pl.program_id(1)
    step = p * NB + i
    slot = lax.rem(step, 2)
    nslot = lax.rem(step + 1, 2)

    def issue(phase, blk, dst_slot):
        rows = pl.ds(blk * BR, BR)

        @pl.when(phase == 0)
        def _():
            pltpu.make_async_copy(pinv0_ref.at[rows, :],
                                  buf0_ref.at[dst_slot], sem.at[0, dst_slot]).start()
            pltpu.make_async_copy(pinv1_ref.at[rows, :],
                                  buf1_ref.at[dst_slot], sem.at[1, dst_slot]).start()

        @pl.when(phase == 1)
        def _():
            pltpu.make_async_copy(phi0_ref.at[rows, :],
                                  buf0_ref.at[dst_slot], sem.at[0, dst_slot]).start()
            pltpu.make_async_copy(phi1_ref.at[rows, :],
                                  buf1_ref.at[dst_slot], sem.at[1, dst_slot]).start()

    @pl.when(step == 0)
    def _():
        issue(0, 0, 0)
        xpt0_ref[...] = lax.dot_general(
            w0_ref[...], x_ref[...], (((0,), (1,)), ((), ())),
            preferred_element_type=jnp.float32)
        xpt1_ref[...] = lax.dot_general(
            w1_ref[...], x_ref[...], (((0,), (1,)), ((), ())),
            preferred_element_type=jnp.float32)

    @pl.when(step < 2 * NB - 1)
    def _():
        nstep = step + 1
        issue(nstep // NB, lax.rem(nstep, NB), nslot)

    pltpu.make_async_copy(pinv0_ref.at[pl.ds(0, BR), :],
                          buf0_ref.at[slot], sem.at[0, slot]).wait()
    pltpu.make_async_copy(pinv1_ref.at[pl.ds(0, BR), :],
                          buf1_ref.at[slot], sem.at[1, slot]).wait()

    cols = pl.ds(i * BR, BR)

    @pl.when(p == 0)
    def _():
        t0 = lax.dot_general(xpt0_ref[...], buf0_ref[slot], _DN_NT,
                             preferred_element_type=jnp.float32)
        yt0_ref[:, cols] = k0_ref[:, cols] * t0
        t1 = lax.dot_general(xpt1_ref[...], buf1_ref[slot], _DN_NT,
                             preferred_element_type=jnp.float32)
        yt1_ref[:, cols] = k1_ref[:, cols] * t1

    @pl.when(p == 1)
    def _():
        z0 = lax.dot_general(yt0_ref[...], buf0_ref[slot], _DN_NT,
                             preferred_element_type=jnp.float32)
        out_ref[0, :, :] = jnp.maximum(z0, 0.0).T
        z1 = lax.dot_general(yt1_ref[...], buf1_ref[slot], _DN_NT,
                             preferred_element_type=jnp.float32)
        out_ref[1, :, :] = jnp.maximum(z1, 0.0).T


def kernel(x, phi_inv_0, phi_0, phi_inv_1, phi_1, W0, W1, k0, k1):
    full = lambda shape: pl.BlockSpec(shape, lambda p, i: (0,) * len(shape))
    hbm = pl.BlockSpec(memory_space=pl.ANY)

    return pl.pallas_call(
        _body,
        grid=(2, NB),
        in_specs=[
            full((N, D)),
            hbm,
            hbm,
            hbm,
            hbm,
            full((D, D)),
            full((D, D)),
            full((1, N)),
            full((1, N)),
        ],
        out_specs=pl.BlockSpec((2, BR, D),
                               lambda p, i: (0, jnp.where(p == 1, i, 0), 0)),
        out_shape=jax.ShapeDtypeStruct((2, N, D), jnp.float32),
        scratch_shapes=[
            pltpu.VMEM((2, BR, N), jnp.float32),
            pltpu.VMEM((2, BR, N), jnp.float32),
            pltpu.VMEM((D, N), jnp.float32),
            pltpu.VMEM((D, N), jnp.float32),
            pltpu.VMEM((D, N), jnp.float32),
            pltpu.VMEM((D, N), jnp.float32),
            pltpu.SemaphoreType.DMA((2, 2)),
        ],
    )(x, phi_inv_0, phi_inv_1, phi_0, phi_1, W0, W1,
      k0.reshape(1, N), k1.reshape(1, N))
